# Initial kernel scaffold; baseline (speedup 1.0000x reference)
import jax
import jax.numpy as jnp
from jax.experimental import pallas as pl


def kernel(m_node, d_node, node_adj, rel_adj):
    raise NotImplementedError("write your pallas kernel here")


# speedup vs baseline: 3.7700x; 3.7700x over previous
import jax
import jax.numpy as jnp
import numpy as np
from jax.experimental import pallas as pl
from jax.experimental.pallas import tpu as pltpu
from jax.experimental.pallas import tpu_sc as plsc

_N = 4096
_B = 2048
_R = 128
_T = _N // _R
_K1 = 8
_K2 = 4


def _threefry2x32(k0, k1, x0, x1):
    ks2 = k0 ^ k1 ^ np.int32(0x1BD11BDA)
    ks = [k0, k1, ks2]
    rot = [[13, 15, 26, 6], [17, 29, 16, 24]]
    x0 = x0 + ks[0]
    x1 = x1 + ks[1]
    for i in range(5):
        for r in rot[i % 2]:
            x0 = x0 + x1
            x1 = (x1 << r) | jax.lax.shift_right_logical(x1, 32 - r)
            x1 = x0 ^ x1
        x0 = x0 + ks[(i + 1) % 3]
        x1 = x1 + ks[(i + 2) % 3] + np.int32(i + 1)
    return x0, x1


def _sample_body(md_ref, dm_ref, starts_ref, key_ref,
                 adj_ref, rel_ref, out_ref, s1, s2):
    i = pl.program_id(0)
    r0 = i * _R
    col = jax.lax.broadcasted_iota(jnp.int32, (_R, _N), 1)
    row = jax.lax.broadcasted_iota(jnp.int32, (_R, _N), 0)
    flat = (r0 + row) * _N + col
    adj = adj_ref[...]
    for kslot, sref in ((0, s1), (1, s2)):
        b0, b1 = _threefry2x32(key_ref[2 * kslot], key_ref[2 * kslot + 1],
                               jnp.zeros_like(flat), flat)
        score = jax.lax.shift_right_logical(b0 ^ b1, 9)
        sref[...] = jnp.where(adj != 0.0, score, jnp.int32(-1))
    lane = jax.lax.broadcasted_iota(jnp.int32, (1, _N), 1)

    def _kill(j, carry):
        rl = md_ref[j] - r0
        c = dm_ref[j]
        v1 = s1[pl.ds(rl, 1), :]
        s1[pl.ds(rl, 1), :] = jnp.where(lane == c, jnp.int32(-1), v1)
        v2 = s2[pl.ds(rl, 1), :]
        s2[pl.ds(rl, 1), :] = jnp.where(lane == c, jnp.int32(-1), v2)
        return carry

    jax.lax.fori_loop(starts_ref[i], starts_ref[i + 1], _kill, 0)
    rel = rel_ref[...]
    big = jnp.int32(2**30)
    cols = []
    for sref, k in ((s1, _K1), (s2, _K2)):
        v = sref[...]
        idx_cols, rel_cols = [], []
        for _ in range(k):
            m = jnp.max(v, axis=1, keepdims=True)
            idxj = jnp.min(jnp.where(v == m, col, big), axis=1, keepdims=True)
            hit = col == idxj
            relj = jnp.min(jnp.where(hit, rel, big), axis=1, keepdims=True)
            idx_cols.append(idxj)
            rel_cols.append(relj - 1)
            v = jnp.where(hit, jnp.int32(-2), v)
        cols += idx_cols + rel_cols
    cols.append(jnp.zeros((_R, 128 - 2 * (_K1 + _K2)), jnp.int32))
    out_ref[...] = jnp.concatenate(cols, axis=1)


def _sample(md_s, dm_s, starts, keydata, node_adj, rel_adj):
    return pl.pallas_call(
        _sample_body,
        grid_spec=pltpu.PrefetchScalarGridSpec(
            num_scalar_prefetch=4,
            grid=(_T,),
            in_specs=[
                pl.BlockSpec((_R, _N), lambda i, *_: (i, 0)),
                pl.BlockSpec((_R, _N), lambda i, *_: (i, 0)),
            ],
            out_specs=pl.BlockSpec((_R, 128), lambda i, *_: (i, 0)),
            scratch_shapes=[pltpu.VMEM((_R, _N), jnp.int32),
                            pltpu.VMEM((_R, _N), jnp.int32)],
        ),
        out_shape=jax.ShapeDtypeStruct((_N, 128), jnp.int32),
    )(md_s, dm_s, starts, keydata, node_adj, rel_adj)


_SC_W = 128


def _sc_row_gather(tbl, idx_flat):
    m = idx_flat.shape[0]
    mesh = plsc.VectorSubcoreMesh(core_axis_name="c", subcore_axis_name="s")

    @pl.kernel(out_type=jax.ShapeDtypeStruct((m, 128), jnp.int32), mesh=mesh)
    def _gather(tbl_hbm, i_hbm, o_hbm):
        def body(i_vmem, o_vmem):
            pltpu.sync_copy(tbl_hbm.at[i_vmem.at[0]], o_vmem)

        pltpu.emit_pipeline(
            body,
            grid=(m // _SC_W,),
            in_specs=[pl.BlockSpec((1, _SC_W), lambda i: (0, i))],
            out_specs=[pl.BlockSpec((_SC_W, 128), lambda i: (i, 0))],
            core_axis_name="s",
            dimension_semantics=(pltpu.PARALLEL,),
        )(i_hbm, o_hbm)

    return _gather(tbl, idx_flat.reshape(1, m))


def kernel(m_node, d_node, node_adj, rel_adj):
    md = jnp.concatenate([m_node, d_node]).astype(jnp.int32)
    dm = jnp.concatenate([d_node, m_node]).astype(jnp.int32)
    order = jnp.argsort(md)
    md_s = md[order]
    dm_s = dm[order]
    bounds = jnp.arange(_T + 1, dtype=jnp.int32) * _R
    starts = jnp.searchsorted(md_s, bounds, side="left").astype(jnp.int32)
    key = jax.random.key(42)
    key, sk1 = jax.random.split(key)
    key, sk2 = jax.random.split(key)
    keydata = jax.lax.bitcast_convert_type(
        jnp.concatenate([jax.random.key_data(sk1), jax.random.key_data(sk2)]),
        jnp.int32)

    tbl = _sample(md_s, dm_s, starts, keydata, node_adj, rel_adj)

    rows1 = _sc_row_gather(tbl, md)
    nei1 = rows1[:, 0:8]
    rows2 = _sc_row_gather(tbl, nei1.reshape(-1))

    mnei1, dnei1 = nei1[:_B], nei1[_B:]
    mrel1, drel1 = rows1[:_B, 8:16], rows1[_B:, 8:16]
    mnei2 = rows2[: _B * _K1, 16:20].reshape(_B, _K1 * _K2)
    dnei2 = rows2[_B * _K1:, 16:20].reshape(_B, _K1 * _K2)
    mrel2 = rows2[: _B * _K1, 20:24].reshape(_B, _K1 * _K2)
    drel2 = rows2[_B * _K1:, 20:24].reshape(_B, _K1 * _K2)

    return (m_node[:, None], mnei1, mnei2, mrel1, mrel2,
            d_node[:, None], dnei1, dnei2, drel1, drel2)


# speedup vs baseline: 4.1162x; 1.0918x over previous
import jax
import jax.numpy as jnp
import numpy as np
from jax.experimental import pallas as pl
from jax.experimental.pallas import tpu as pltpu
from jax.experimental.pallas import tpu_sc as plsc

_N = 4096
_B = 2048
_R = 128
_T = _N // _R
_K1 = 8
_K2 = 4


def _threefry2x32(k0, k1, x0, x1):
    ks2 = k0 ^ k1 ^ np.int32(0x1BD11BDA)
    ks = [k0, k1, ks2]
    rot = [[13, 15, 26, 6], [17, 29, 16, 24]]
    x0 = x0 + ks[0]
    x1 = x1 + ks[1]
    for i in range(5):
        for r in rot[i % 2]:
            x0 = x0 + x1
            x1 = (x1 << r) | jax.lax.shift_right_logical(x1, 32 - r)
            x1 = x0 ^ x1
        x0 = x0 + ks[(i + 1) % 3]
        x1 = x1 + ks[(i + 2) % 3] + np.int32(i + 1)
    return x0, x1


def _sample_body(md_ref, dm_ref, starts_ref, key_ref,
                 adj_ref, rel_ref, out_ref, s1, s2):
    i = pl.program_id(0)
    r0 = i * _R
    col = jax.lax.broadcasted_iota(jnp.int32, (_R, _N), 1)
    row = jax.lax.broadcasted_iota(jnp.int32, (_R, _N), 0)
    flat = (r0 + row) * _N + col
    adj = adj_ref[...]
    for kslot, sref in ((0, s1), (1, s2)):
        b0, b1 = _threefry2x32(key_ref[2 * kslot], key_ref[2 * kslot + 1],
                               jnp.zeros_like(flat), flat)
        score = jax.lax.shift_right_logical(b0 ^ b1, 9)
        sref[...] = jnp.where(adj != 0.0, score, jnp.int32(-1))
    lane = jax.lax.broadcasted_iota(jnp.int32, (1, _N), 1)

    def _kill(j, carry):
        rl = md_ref[j] - r0
        c = dm_ref[j]
        v1 = s1[pl.ds(rl, 1), :]
        s1[pl.ds(rl, 1), :] = jnp.where(lane == c, jnp.int32(-1), v1)
        v2 = s2[pl.ds(rl, 1), :]
        s2[pl.ds(rl, 1), :] = jnp.where(lane == c, jnp.int32(-1), v2)
        return carry

    jax.lax.fori_loop(starts_ref[i], starts_ref[i + 1], _kill, 0)
    aux = (col << 4) | rel_ref[...]
    big = jnp.int32(2**30)
    cols = []
    for sref, k in ((s1, _K1), (s2, _K2)):
        v = sref[...]
        idx_cols, rel_cols = [], []
        for _ in range(k):
            m = jnp.max(v, axis=1, keepdims=True)
            auxj = jnp.min(jnp.where(v == m, aux, big), axis=1, keepdims=True)
            idx_cols.append(jax.lax.shift_right_logical(auxj, 4))
            rel_cols.append((auxj & 15) - 1)
            v = jnp.where(aux == auxj, jnp.int32(-2), v)
        cols += idx_cols + rel_cols
    cols.append(jnp.zeros((_R, 128 - 2 * (_K1 + _K2)), jnp.int32))
    out_ref[...] = jnp.concatenate(cols, axis=1)


def _sample(md_s, dm_s, starts, keydata, node_adj, rel_adj):
    return pl.pallas_call(
        _sample_body,
        grid_spec=pltpu.PrefetchScalarGridSpec(
            num_scalar_prefetch=4,
            grid=(_T,),
            in_specs=[
                pl.BlockSpec((_R, _N), lambda i, *_: (i, 0)),
                pl.BlockSpec((_R, _N), lambda i, *_: (i, 0)),
            ],
            out_specs=pl.BlockSpec((_R, 128), lambda i, *_: (i, 0)),
            scratch_shapes=[pltpu.VMEM((_R, _N), jnp.int32),
                            pltpu.VMEM((_R, _N), jnp.int32)],
        ),
        out_shape=jax.ShapeDtypeStruct((_N, 128), jnp.int32),
        compiler_params=pltpu.CompilerParams(
            dimension_semantics=("parallel",)),
    )(md_s, dm_s, starts, keydata, node_adj, rel_adj)


_SC_W = 128


def _sc_row_gather(tbl, idx_flat):
    m = idx_flat.shape[0]
    mesh = plsc.VectorSubcoreMesh(core_axis_name="c", subcore_axis_name="s")

    @pl.kernel(out_type=jax.ShapeDtypeStruct((m, 128), jnp.int32), mesh=mesh)
    def _gather(tbl_hbm, i_hbm, o_hbm):
        def body(i_vmem, o_vmem):
            pltpu.sync_copy(tbl_hbm.at[i_vmem.at[0]], o_vmem)

        pltpu.emit_pipeline(
            body,
            grid=(m // _SC_W,),
            in_specs=[pl.BlockSpec((1, _SC_W), lambda i: (0, i))],
            out_specs=[pl.BlockSpec((_SC_W, 128), lambda i: (i, 0))],
            core_axis_name="s",
            dimension_semantics=(pltpu.PARALLEL,),
        )(i_hbm, o_hbm)

    return _gather(tbl, idx_flat.reshape(1, m))


def kernel(m_node, d_node, node_adj, rel_adj):
    md = jnp.concatenate([m_node, d_node]).astype(jnp.int32)
    dm = jnp.concatenate([d_node, m_node]).astype(jnp.int32)
    order = jnp.argsort(md)
    md_s = md[order]
    dm_s = dm[order]
    bounds = jnp.arange(_T + 1, dtype=jnp.int32) * _R
    starts = jnp.searchsorted(md_s, bounds, side="left").astype(jnp.int32)
    key = jax.random.key(42)
    key, sk1 = jax.random.split(key)
    key, sk2 = jax.random.split(key)
    keydata = jax.lax.bitcast_convert_type(
        jnp.concatenate([jax.random.key_data(sk1), jax.random.key_data(sk2)]),
        jnp.int32)

    tbl = _sample(md_s, dm_s, starts, keydata, node_adj, rel_adj)

    rows1 = _sc_row_gather(tbl, md)
    nei1 = rows1[:, 0:8]
    rows2 = _sc_row_gather(tbl, nei1.reshape(-1))

    mnei1, dnei1 = nei1[:_B], nei1[_B:]
    mrel1, drel1 = rows1[:_B, 8:16], rows1[_B:, 8:16]
    mnei2 = rows2[: _B * _K1, 16:20].reshape(_B, _K1 * _K2)
    dnei2 = rows2[_B * _K1:, 16:20].reshape(_B, _K1 * _K2)
    mrel2 = rows2[: _B * _K1, 20:24].reshape(_B, _K1 * _K2)
    drel2 = rows2[_B * _K1:, 20:24].reshape(_B, _K1 * _K2)

    return (m_node[:, None], mnei1, mnei2, mrel1, mrel2,
            d_node[:, None], dnei1, dnei2, drel1, drel2)


# speedup vs baseline: 4.2516x; 1.0329x over previous
import jax
import jax.numpy as jnp
import numpy as np
from jax.experimental import pallas as pl
from jax.experimental.pallas import tpu as pltpu
from jax.experimental.pallas import tpu_sc as plsc

_N = 4096
_B = 2048
_R = 128
_T = _N // _R
_K1 = 8
_K2 = 4


def _threefry2x32(k0, k1, x0, x1):
    ks2 = k0 ^ k1 ^ np.int32(0x1BD11BDA)
    ks = [k0, k1, ks2]
    rot = [[13, 15, 26, 6], [17, 29, 16, 24]]
    x0 = x0 + ks[0]
    x1 = x1 + ks[1]
    for i in range(5):
        for r in rot[i % 2]:
            x0 = x0 + x1
            x1 = (x1 << r) | jax.lax.shift_right_logical(x1, 32 - r)
            x1 = x0 ^ x1
        x0 = x0 + ks[(i + 1) % 3]
        x1 = x1 + ks[(i + 2) % 3] + np.int32(i + 1)
    return x0, x1


_P = 256


def _sample_body(starts_ref, key_ref,
                 mdc_ref, dmt_ref, adj_ref, rel_ref, out_ref, s1, s2, ka):
    i = pl.program_id(0)
    r0 = i * _R
    col = jax.lax.broadcasted_iota(jnp.int32, (_R, _N), 1)
    row = jax.lax.broadcasted_iota(jnp.int32, (_R, _N), 0)
    flat = (r0 + row) * _N + col
    ka[...] = jnp.zeros((_R, _N), jnp.float32)
    rowv = r0 + jax.lax.broadcasted_iota(jnp.int32, (_R, 1), 0)
    lane = jax.lax.broadcasted_iota(jnp.int32, (1, _N), 1)

    def _kill_chunk(c, carry):
        mdc = mdc_ref[pl.ds(c, 1), :]
        dmt = dmt_ref[pl.ds(c * _P, _P), :]
        a_t = (mdc == rowv).astype(jnp.bfloat16)
        b = (dmt == lane).astype(jnp.bfloat16)
        ka[...] += jax.lax.dot_general(
            a_t, b, (((1,), (0,)), ((), ())),
            preferred_element_type=jnp.float32)
        return carry

    cs = starts_ref[i] // _P
    ce = (starts_ref[i + 1] + (_P - 1)) // _P
    jax.lax.fori_loop(cs, ce, _kill_chunk, 0)
    allowed = jnp.logical_and(adj_ref[...] != 0.0, ka[...] == 0.0)
    for kslot, sref in ((0, s1), (1, s2)):
        b0, b1 = _threefry2x32(key_ref[2 * kslot], key_ref[2 * kslot + 1],
                               jnp.zeros_like(flat), flat)
        score = jax.lax.shift_right_logical(b0 ^ b1, 9)
        sref[...] = jnp.where(allowed, score, jnp.int32(-1))
    aux = (col << 4) | rel_ref[...]
    big = jnp.int32(2**30)
    cols = []
    for sref, k in ((s1, _K1), (s2, _K2)):
        v = sref[...]
        idx_cols, rel_cols = [], []
        for _ in range(k):
            m = jnp.max(v, axis=1, keepdims=True)
            auxj = jnp.min(jnp.where(v == m, aux, big), axis=1, keepdims=True)
            idx_cols.append(jax.lax.shift_right_logical(auxj, 4))
            rel_cols.append((auxj & 15) - 1)
            v = jnp.where(aux == auxj, jnp.int32(-2), v)
        cols += idx_cols + rel_cols
    cols.append(jnp.zeros((_R, 128 - 2 * (_K1 + _K2)), jnp.int32))
    out_ref[...] = jnp.concatenate(cols, axis=1)


def _sample(md_s, dm_s, starts, keydata, node_adj, rel_adj):
    n_chunks = 2 * _B // _P
    return pl.pallas_call(
        _sample_body,
        grid_spec=pltpu.PrefetchScalarGridSpec(
            num_scalar_prefetch=2,
            grid=(_T,),
            in_specs=[
                pl.BlockSpec((n_chunks, _P), lambda i, *_: (0, 0)),
                pl.BlockSpec((2 * _B, 1), lambda i, *_: (0, 0)),
                pl.BlockSpec((_R, _N), lambda i, *_: (i, 0)),
                pl.BlockSpec((_R, _N), lambda i, *_: (i, 0)),
            ],
            out_specs=pl.BlockSpec((_R, 128), lambda i, *_: (i, 0)),
            scratch_shapes=[pltpu.VMEM((_R, _N), jnp.int32),
                            pltpu.VMEM((_R, _N), jnp.int32),
                            pltpu.VMEM((_R, _N), jnp.float32)],
        ),
        out_shape=jax.ShapeDtypeStruct((_N, 128), jnp.int32),
        compiler_params=pltpu.CompilerParams(
            dimension_semantics=("parallel",)),
    )(starts, keydata, md_s.reshape(n_chunks, _P), dm_s.reshape(2 * _B, 1),
      node_adj, rel_adj)


_SC_W = 128


def _sc_row_gather(tbl, idx_flat):
    m = idx_flat.shape[0]
    mesh = plsc.VectorSubcoreMesh(core_axis_name="c", subcore_axis_name="s")

    @pl.kernel(out_type=jax.ShapeDtypeStruct((m, 128), jnp.int32), mesh=mesh)
    def _gather(tbl_hbm, i_hbm, o_hbm):
        def body(i_vmem, o_vmem):
            pltpu.sync_copy(tbl_hbm.at[i_vmem.at[0]], o_vmem)

        pltpu.emit_pipeline(
            body,
            grid=(m // _SC_W,),
            in_specs=[pl.BlockSpec((1, _SC_W), lambda i: (0, i))],
            out_specs=[pl.BlockSpec((_SC_W, 128), lambda i: (i, 0))],
            core_axis_name="s",
            dimension_semantics=(pltpu.PARALLEL,),
        )(i_hbm, o_hbm)

    return _gather(tbl, idx_flat.reshape(1, m))


def kernel(m_node, d_node, node_adj, rel_adj):
    md = jnp.concatenate([m_node, d_node]).astype(jnp.int32)
    dm = jnp.concatenate([d_node, m_node]).astype(jnp.int32)
    order = jnp.argsort(md)
    md_s = md[order]
    dm_s = dm[order]
    bounds = jnp.arange(_T + 1, dtype=jnp.int32) * _R
    starts = jnp.searchsorted(md_s, bounds, side="left").astype(jnp.int32)
    key = jax.random.key(42)
    key, sk1 = jax.random.split(key)
    key, sk2 = jax.random.split(key)
    keydata = jax.lax.bitcast_convert_type(
        jnp.concatenate([jax.random.key_data(sk1), jax.random.key_data(sk2)]),
        jnp.int32)

    tbl = _sample(md_s, dm_s, starts, keydata, node_adj, rel_adj)

    rows1 = _sc_row_gather(tbl, md)
    nei1 = rows1[:, 0:8]
    rows2 = _sc_row_gather(tbl, nei1.reshape(-1))

    mnei1, dnei1 = nei1[:_B], nei1[_B:]
    mrel1, drel1 = rows1[:_B, 8:16], rows1[_B:, 8:16]
    mnei2 = rows2[: _B * _K1, 16:20].reshape(_B, _K1 * _K2)
    dnei2 = rows2[_B * _K1:, 16:20].reshape(_B, _K1 * _K2)
    mrel2 = rows2[: _B * _K1, 20:24].reshape(_B, _K1 * _K2)
    drel2 = rows2[_B * _K1:, 20:24].reshape(_B, _K1 * _K2)

    return (m_node[:, None], mnei1, mnei2, mrel1, mrel2,
            d_node[:, None], dnei1, dnei2, drel1, drel2)


# speedup vs baseline: 4.2776x; 1.0061x over previous
import functools

import jax
import jax.numpy as jnp
import numpy as np
from jax.experimental import pallas as pl
from jax.experimental.pallas import tpu as pltpu
from jax.experimental.pallas import tpu_sc as plsc

_N = 4096
_B = 2048
_R = 128
_T = _N // _R
_K1 = 8
_K2 = 4


def _threefry2x32(k0, k1, x0, x1):
    ks2 = k0 ^ k1 ^ np.int32(0x1BD11BDA)
    ks = [k0, k1, ks2]
    rot = [[13, 15, 26, 6], [17, 29, 16, 24]]
    x0 = x0 + ks[0]
    x1 = x1 + ks[1]
    for i in range(5):
        for r in rot[i % 2]:
            x0 = x0 + x1
            x1 = (x1 << r) | jax.lax.shift_right_logical(x1, 32 - r)
            x1 = x0 ^ x1
        x0 = x0 + ks[(i + 1) % 3]
        x1 = x1 + ks[(i + 2) % 3] + np.int32(i + 1)
    return x0, x1


_P = 256


def _sample_body(base_ref, starts_ref, key_ref,
                 mdc_ref, dmt_ref, adj_ref, rel_ref, out_ref, s1, s2, ka):
    i = pl.program_id(0)
    r0 = base_ref[0] + i * _R
    g = base_ref[1] + i
    col = jax.lax.broadcasted_iota(jnp.int32, (_R, _N), 1)
    row = jax.lax.broadcasted_iota(jnp.int32, (_R, _N), 0)
    flat = (r0 + row) * _N + col
    ka[...] = jnp.zeros((_R, _N), jnp.float32)
    rowv = r0 + jax.lax.broadcasted_iota(jnp.int32, (_R, 1), 0)
    lane = jax.lax.broadcasted_iota(jnp.int32, (1, _N), 1)

    def _kill_chunk(c, carry):
        mdc = mdc_ref[pl.ds(c, 1), :]
        dmt = dmt_ref[pl.ds(c * _P, _P), :]
        a_t = (mdc == rowv).astype(jnp.bfloat16)
        b = (dmt == lane).astype(jnp.bfloat16)
        ka[...] += jax.lax.dot_general(
            a_t, b, (((1,), (0,)), ((), ())),
            preferred_element_type=jnp.float32)
        return carry

    cs = starts_ref[g] // _P
    ce = (starts_ref[g + 1] + (_P - 1)) // _P
    jax.lax.fori_loop(cs, ce, _kill_chunk, 0)
    allowed = jnp.logical_and(adj_ref[...] != 0.0, ka[...] == 0.0)
    for kslot, sref in ((0, s1), (1, s2)):
        b0, b1 = _threefry2x32(key_ref[2 * kslot], key_ref[2 * kslot + 1],
                               jnp.zeros_like(flat), flat)
        score = jax.lax.shift_right_logical(b0 ^ b1, 9)
        sref[...] = jnp.where(allowed, score, jnp.int32(-1))
    aux = (col << 4) | rel_ref[...]
    big = jnp.int32(2**30)
    cols = []
    for sref, k in ((s1, _K1), (s2, _K2)):
        v = sref[...]
        idx_cols, rel_cols = [], []
        for _ in range(k):
            m = jnp.max(v, axis=1, keepdims=True)
            auxj = jnp.min(jnp.where(v == m, aux, big), axis=1, keepdims=True)
            idx_cols.append(jax.lax.shift_right_logical(auxj, 4))
            rel_cols.append((auxj & 15) - 1)
            v = jnp.where(aux == auxj, jnp.int32(-2), v)
        cols += idx_cols + rel_cols
    cols.append(jnp.zeros((_R, 128 - 2 * (_K1 + _K2)), jnp.int32))
    out_ref[...] = jnp.concatenate(cols, axis=1)


def _sample(base, starts, keydata, mdc, dmt, adj_shard, rel_shard):
    n_rows = adj_shard.shape[0]
    n_chunks = 2 * _B // _P
    return pl.pallas_call(
        _sample_body,
        grid_spec=pltpu.PrefetchScalarGridSpec(
            num_scalar_prefetch=3,
            grid=(n_rows // _R,),
            in_specs=[
                pl.BlockSpec((n_chunks, _P), lambda i, *_: (0, 0)),
                pl.BlockSpec((2 * _B, 1), lambda i, *_: (0, 0)),
                pl.BlockSpec((_R, _N), lambda i, *_: (i, 0)),
                pl.BlockSpec((_R, _N), lambda i, *_: (i, 0)),
            ],
            out_specs=pl.BlockSpec((_R, 128), lambda i, *_: (i, 0)),
            scratch_shapes=[pltpu.VMEM((_R, _N), jnp.int32),
                            pltpu.VMEM((_R, _N), jnp.int32),
                            pltpu.VMEM((_R, _N), jnp.float32)],
        ),
        out_shape=jax.ShapeDtypeStruct((n_rows, 128), jnp.int32),
        compiler_params=pltpu.CompilerParams(
            dimension_semantics=("parallel",)),
    )(base, starts, keydata, mdc, dmt, adj_shard, rel_shard)


_SC_W = 128


def _sc_row_gather(tbl, idx_flat):
    m = idx_flat.shape[0]
    mesh = plsc.VectorSubcoreMesh(core_axis_name="c", subcore_axis_name="s")

    @pl.kernel(out_type=jax.ShapeDtypeStruct((m, 128), jnp.int32), mesh=mesh)
    def _gather(tbl_hbm, i_hbm, o_hbm):
        def body(i_vmem, o_vmem):
            pltpu.sync_copy(tbl_hbm.at[i_vmem.at[0]], o_vmem)

        pltpu.emit_pipeline(
            body,
            grid=(m // _SC_W,),
            in_specs=[pl.BlockSpec((1, _SC_W), lambda i: (0, i))],
            out_specs=[pl.BlockSpec((_SC_W, 128), lambda i: (i, 0))],
            core_axis_name="s",
            dimension_semantics=(pltpu.PARALLEL,),
        )(i_hbm, o_hbm)

    return _gather(tbl, idx_flat.reshape(1, m))


def kernel(m_node, d_node, node_adj, rel_adj):
    md = jnp.concatenate([m_node, d_node]).astype(jnp.int32)
    dm = jnp.concatenate([d_node, m_node]).astype(jnp.int32)
    order = jnp.argsort(md)
    md_s = md[order]
    dm_s = dm[order]
    bounds = jnp.arange(_T + 1, dtype=jnp.int32) * _R
    starts = jnp.searchsorted(md_s, bounds, side="left").astype(jnp.int32)
    key = jax.random.key(42)
    key, sk1 = jax.random.split(key)
    key, sk2 = jax.random.split(key)
    keydata = jax.lax.bitcast_convert_type(
        jnp.concatenate([jax.random.key_data(sk1), jax.random.key_data(sk2)]),
        jnp.int32)
    n_chunks = 2 * _B // _P
    mdc = md_s.reshape(n_chunks, _P)
    dmt = dm_s.reshape(2 * _B, 1)

    devs = jax.devices()
    ndev = 2 if len(devs) >= 2 else 1
    mesh = jax.sharding.Mesh(np.array(devs[:ndev]), ("x",))
    ps = jax.sharding.PartitionSpec

    @functools.partial(
        jax.shard_map, mesh=mesh, check_vma=False,
        in_specs=(ps(), ps(), ps(), ps(), ps(),
                  ps("x", None), ps("x", None)),
        out_specs=(ps("x", None), ps("x", None)))
    def _run(q, starts_, keydata_, mdc_, dmt_, adj_sh, rel_sh):
        dev = jax.lax.axis_index("x").astype(jnp.int32)
        base = jnp.stack([dev * jnp.int32(_N // ndev),
                          dev * jnp.int32(_T // ndev)])
        tbl_loc = _sample(base, starts_, keydata_, mdc_, dmt_, adj_sh, rel_sh)
        tbl = jax.lax.all_gather(tbl_loc, "x", axis=0, tiled=True)
        q_loc = jax.lax.dynamic_slice_in_dim(q, dev * (2 * _B // ndev),
                                             2 * _B // ndev)
        rows1_loc = _sc_row_gather(tbl, q_loc)
        rows2_loc = _sc_row_gather(tbl, rows1_loc[:, 0:8].reshape(-1))
        return rows1_loc, rows2_loc

    rows1, rows2 = _run(md, starts, keydata, mdc, dmt, node_adj, rel_adj)
    nei1 = rows1[:, 0:8]

    mnei1, dnei1 = nei1[:_B], nei1[_B:]
    mrel1, drel1 = rows1[:_B, 8:16], rows1[_B:, 8:16]
    mnei2 = rows2[: _B * _K1, 16:20].reshape(_B, _K1 * _K2)
    dnei2 = rows2[_B * _K1:, 16:20].reshape(_B, _K1 * _K2)
    mrel2 = rows2[: _B * _K1, 20:24].reshape(_B, _K1 * _K2)
    drel2 = rows2[_B * _K1:, 20:24].reshape(_B, _K1 * _K2)

    return (m_node[:, None], mnei1, mnei2, mrel1, mrel2,
            d_node[:, None], dnei1, dnei2, drel1, drel2)
